# baseline (speedup 1.0000x reference)
import jax
import jax.numpy as jnp
from jax import lax
from jax.experimental import pallas as pl
from jax.experimental.pallas import tpu as pltpu
from jax.experimental.pallas import tpu_sc as plsc

_NUM_CLASSES = 100000
_DIM = 64
_BATCH = 4096
_NUM_TEMPLATES = 200

_NC = 2
_NS = 16
_NW = _NC * _NS
_N = _BATCH * _NUM_TEMPLATES
_BPW = _N // _NW
_C = 128
_C2 = _C * _DIM // 128
_NCHUNK = _BPW // _C
_NBUF = 4
_AHEAD = 2
_RW = 8


def _sc_body(emb, perm, labels2, feat, out,
             lab_v, map_v, rows_v, feat_v,
             remap_sem, rows_sem, feat_sem, out_sem):
    wid = lax.axis_index("s") * _NC + lax.axis_index("c")
    base = wid * _BPW
    base2 = wid * (_BPW * _DIM // 128)

    pltpu.sync_copy(labels2.at[pl.ds(wid * _NCHUNK, _NCHUNK)], lab_v)

    def remap_issue(k, carry):
        pltpu.async_copy(perm.at[lab_v.at[k]], map_v.at[k], remap_sem)

        @pl.when(k >= _RW)
        def _():
            pltpu.make_async_copy(
                perm.at[lab_v.at[k]], map_v.at[k], remap_sem).wait()
        return carry

    lax.fori_loop(0, _NCHUNK, remap_issue, 0)

    def remap_drain(k, carry):
        pltpu.make_async_copy(
            perm.at[lab_v.at[k]], map_v.at[k], remap_sem).wait()
        return carry

    lax.fori_loop(0, _RW, remap_drain, 0)

    def issue(h):
        t = h % _NBUF
        pltpu.async_copy(emb.at[map_v.at[h]], rows_v.at[t], rows_sem.at[t])
        pltpu.async_copy(feat.at[pl.ds(base2 + h * _C2, _C2)], feat_v.at[t],
                         feat_sem.at[t])

    for h in range(_AHEAD):
        issue(h)

    def chunk(g, carry):
        s = g % _NBUF
        off2 = base2 + g * _C2
        pltpu.make_async_copy(
            emb.at[map_v.at[g]], rows_v.at[s], rows_sem.at[s]).wait()
        pltpu.make_async_copy(
            feat.at[pl.ds(off2, _C2)], feat_v.at[s], feat_sem.at[s]).wait()

        def addrow(j2, c2):
            for k in range(8):
                sl = pl.ds(k * 16, 16)
                feat_v[s, j2, sl] = (
                    feat_v[s, j2, sl]
                    + rows_v[s, 2 * j2 + k // 4, pl.ds((k % 4) * 16, 16)])
            return c2

        lax.fori_loop(0, _C2, addrow, 0)
        pltpu.async_copy(feat_v.at[s], out.at[pl.ds(off2, _C2)], out_sem.at[s])

        h = g + _AHEAD

        @pl.when(h < _NCHUNK)
        def _():
            t = h % _NBUF
            hoff2 = base2 + h * _C2

            @pl.when(h >= _NBUF)
            def _():
                pltpu.make_async_copy(
                    feat_v.at[t], out.at[pl.ds(hoff2, _C2)], out_sem.at[t]
                ).wait()

            pltpu.async_copy(emb.at[map_v.at[h]], rows_v.at[t],
                             rows_sem.at[t])
            pltpu.async_copy(feat.at[pl.ds(hoff2, _C2)], feat_v.at[t],
                             feat_sem.at[t])
        return carry

    lax.fori_loop(0, _NCHUNK, chunk, 0)

    def drain(g, carry):
        s = g % _NBUF
        off2 = base2 + g * _C2
        pltpu.make_async_copy(
            feat_v.at[s], out.at[pl.ds(off2, _C2)], out_sem.at[s]).wait()
        return carry

    lax.fori_loop(_NCHUNK - _NBUF, _NCHUNK, drain, 0)


def kernel(template_features, template_labels, embeddings):
    perm = jax.random.permutation(
        jax.random.key(42), embeddings.shape[0]).astype(jnp.int32)
    labels2 = template_labels.reshape(_N // _C, _C).astype(jnp.int32)
    feat = template_features.reshape(_N * _DIM // 128, 128)
    mesh = plsc.VectorSubcoreMesh(core_axis_name="c", subcore_axis_name="s")
    run = pl.kernel(
        _sc_body,
        out_type=jax.ShapeDtypeStruct((_N * _DIM // 128, 128), jnp.float32),
        mesh=mesh,
        scratch_types=[
            pltpu.VMEM((_NCHUNK, _C), jnp.int32),
            pltpu.VMEM((_NCHUNK, _C), jnp.int32),
            pltpu.VMEM((_NBUF, _C, _DIM), jnp.float32),
            pltpu.VMEM((_NBUF, _C2, 128), jnp.float32),
            pltpu.SemaphoreType.DMA,
            pltpu.SemaphoreType.DMA((_NBUF,)),
            pltpu.SemaphoreType.DMA((_NBUF,)),
            pltpu.SemaphoreType.DMA((_NBUF,)),
        ],
        compiler_params=pltpu.CompilerParams(use_tc_tiling_on_sc=False),
    )
    out = run(embeddings, perm, labels2, feat)
    return out.reshape(_BATCH, _NUM_TEMPLATES, _DIM)


# speedup vs baseline: 1.3577x; 1.3577x over previous
import numpy as np

import jax
import jax.numpy as jnp
from jax import lax
from jax.experimental import pallas as pl
from jax.experimental.pallas import tpu as pltpu
from jax.experimental.pallas import tpu_sc as plsc

_NUM_CLASSES = 100000
_DIM = 64
_BATCH = 4096
_NUM_TEMPLATES = 200

_NC = 2
_NS = 16
_NW = _NC * _NS
_N = _BATCH * _NUM_TEMPLATES
_BPW = _N // _NW
_C = 128
_NCHUNK = _BPW // _C
_NBUF = 4
_AHEAD = 2
_RW = 8

_PERM_NP = np.asarray(
    jax.random.permutation(jax.random.key(42), _NUM_CLASSES), dtype=np.int32)


def _sc_body(emb, perm, labels2, feat, out,
             lab_v, map_v, feat_v,
             remap_sem, rows_sem, feat_sem, out_sem):
    wid = lax.axis_index("s") * _NC + lax.axis_index("c")
    base = wid * _BPW

    pltpu.sync_copy(labels2.at[pl.ds(wid * _NCHUNK, _NCHUNK)], lab_v)

    def remap_issue(k, carry):
        pltpu.async_copy(perm.at[lab_v.at[k]], map_v.at[k], remap_sem)

        @pl.when(k >= _RW)
        def _():
            pltpu.make_async_copy(
                perm.at[lab_v.at[k]], map_v.at[k], remap_sem).wait()
        return carry

    lax.fori_loop(0, _NCHUNK, remap_issue, 0)

    def remap_drain(k, carry):
        pltpu.make_async_copy(
            perm.at[lab_v.at[k]], map_v.at[k], remap_sem).wait()
        return carry

    lax.fori_loop(0, _RW, remap_drain, 0)

    def issue(h):
        t = h % _NBUF
        pltpu.async_copy(feat.at[pl.ds(base + h * _C, _C)], feat_v.at[t],
                         feat_sem.at[t])

    def issue_gather(h):
        t = h % _NBUF
        pltpu.make_async_copy(
            feat.at[pl.ds(base + h * _C, _C)], feat_v.at[t],
            feat_sem.at[t]).wait()
        pltpu.async_copy(emb.at[map_v.at[h]], feat_v.at[t], rows_sem.at[t],
                         add=True)

    for h in range(_AHEAD):
        issue(h)
    issue_gather(0)

    def chunk(g, carry):
        s = g % _NBUF
        off = base + g * _C

        @pl.when(g + 1 < _NCHUNK)
        def _():
            issue_gather(g + 1)

        pltpu.make_async_copy(
            emb.at[map_v.at[g]], feat_v.at[s], rows_sem.at[s]).wait()
        pltpu.async_copy(feat_v.at[s], out.at[pl.ds(off, _C)], out_sem.at[s])

        h = g + _AHEAD

        @pl.when(h < _NCHUNK)
        def _():
            t = h % _NBUF
            hoff = base + h * _C

            @pl.when(h >= _NBUF)
            def _():
                pltpu.make_async_copy(
                    feat_v.at[t], out.at[pl.ds(hoff, _C)], out_sem.at[t]
                ).wait()

            pltpu.async_copy(feat.at[pl.ds(hoff, _C)], feat_v.at[t],
                             feat_sem.at[t])
        return carry

    lax.fori_loop(0, _NCHUNK, chunk, 0)

    def drain(g, carry):
        s = g % _NBUF
        off = base + g * _C
        pltpu.make_async_copy(
            feat_v.at[s], out.at[pl.ds(off, _C)], out_sem.at[s]).wait()
        return carry

    lax.fori_loop(_NCHUNK - _NBUF, _NCHUNK, drain, 0)


def kernel(template_features, template_labels, embeddings):
    perm = jnp.asarray(_PERM_NP)
    labels2 = template_labels.reshape(_N // _C, _C).astype(jnp.int32)
    feat = template_features.reshape(_N, _DIM)
    mesh = plsc.VectorSubcoreMesh(core_axis_name="c", subcore_axis_name="s")
    run = pl.kernel(
        _sc_body,
        out_type=jax.ShapeDtypeStruct((_N, _DIM), jnp.float32),
        mesh=mesh,
        scratch_types=[
            pltpu.VMEM((_NCHUNK, _C), jnp.int32),
            pltpu.VMEM((_NCHUNK, _C), jnp.int32),
            pltpu.VMEM((_NBUF, _C, _DIM), jnp.float32),
            pltpu.SemaphoreType.DMA,
            pltpu.SemaphoreType.DMA((_NBUF,)),
            pltpu.SemaphoreType.DMA((_NBUF,)),
            pltpu.SemaphoreType.DMA((_NBUF,)),
        ],
        compiler_params=pltpu.CompilerParams(use_tc_tiling_on_sc=False),
    )
    out = run(embeddings, perm, labels2, feat)
    return out.reshape(_BATCH, _NUM_TEMPLATES, _DIM)
